# baseline (speedup 1.0000x reference)
import functools

import jax
import jax.numpy as jnp
from jax import lax
from jax.experimental import pallas as pl
from jax.experimental.pallas import tpu as pltpu
from jax.experimental.pallas import tpu_sc as plsc

_NUM_CLASSES = 100000
_FEAT_DIM = 64
_BATCH = 16384

_INFO = plsc.get_sparse_core_info()
_NC = _INFO.num_cores
_NS = _INFO.num_subcores
_LANES = _INFO.num_lanes
_NW = _NC * _NS
_BPW = _BATCH // _NW
_CHUNK = 128
_NCHUNK = _BPW // _CHUNK

_BLK = 512
_NBLK = _BATCH // _BLK


def _sc_gather_body(labels_hbm, centers_hbm, out_hbm, idx_v, rows_v, gsem):
    wid = lax.axis_index("s") * _NC + lax.axis_index("c")
    base = wid * _BPW

    lcopies = [
        pltpu.async_copy(labels_hbm.at[pl.ds(base + j * _CHUNK, _CHUNK)],
                         idx_v.at[j], gsem.at[j])
        for j in range(_NCHUNK)
    ]
    gathers = []
    for j in range(_NCHUNK):
        lcopies[j].wait()
        gathers.append(
            pltpu.async_copy(centers_hbm.at[idx_v.at[j]],
                             rows_v.at[pl.ds(j * _CHUNK, _CHUNK)],
                             gsem.at[j]))
    for j in range(_NCHUNK):
        gathers[j].wait()
        pltpu.sync_copy(
            rows_v.at[pl.ds(j * _CHUNK, _CHUNK)],
            out_hbm.at[pl.ds(base + j * _CHUNK, _CHUNK), pl.ds(0, _FEAT_DIM)])


def _tc_mse_body(f_ref, g_ref, o_ref):
    step = pl.program_id(0)

    @pl.when(step == 0)
    def _init():
        o_ref[...] = jnp.zeros_like(o_ref)

    d = f_ref[...] - g_ref[:, : _FEAT_DIM]
    o_ref[...] += jnp.sum(d * d).reshape(1, 1)


@functools.partial(jax.jit, static_argnames=())
def kernel(features, labels, centers):
    mesh = plsc.VectorSubcoreMesh(core_axis_name="c", subcore_axis_name="s")
    gathered = pl.kernel(
        _sc_gather_body,
        mesh=mesh,
        out_type=jax.ShapeDtypeStruct((_BATCH, 2 * _FEAT_DIM), jnp.float32),
        scratch_types=[
            pltpu.VMEM((_NCHUNK, _CHUNK), jnp.int32),
            pltpu.VMEM((_BPW, _FEAT_DIM), jnp.float32),
            pltpu.SemaphoreType.DMA((_NCHUNK,)),
        ],
        compiler_params=pltpu.CompilerParams(use_tc_tiling_on_sc=False),
    )(labels.astype(jnp.int32), centers)

    sq = pl.pallas_call(
        _tc_mse_body,
        grid=(_NBLK,),
        out_shape=jax.ShapeDtypeStruct((1, 1), jnp.float32),
        in_specs=[
            pl.BlockSpec((_BLK, _FEAT_DIM), lambda i: (i, 0)),
            pl.BlockSpec((_BLK, 2 * _FEAT_DIM), lambda i: (i, 0)),
        ],
        out_specs=pl.BlockSpec((1, 1), lambda i: (0, 0)),
    )(features, gathered)
    return sq[0, 0] * jnp.float32(1.0 / (_BATCH * _FEAT_DIM))


# speedup vs baseline: 1.1421x; 1.1421x over previous
import functools

import jax
import jax.numpy as jnp
from jax import lax
from jax.experimental import pallas as pl
from jax.experimental.pallas import tpu as pltpu
from jax.experimental.pallas import tpu_sc as plsc

_NUM_CLASSES = 100000
_FEAT_DIM = 64
_BATCH = 16384

_INFO = plsc.get_sparse_core_info()
_NC = _INFO.num_cores
_NS = _INFO.num_subcores
_LANES = _INFO.num_lanes
_NW = _NC * _NS
_BPW = _BATCH // _NW
_CHUNK = 128
_NCHUNK = _BPW // _CHUNK
_GRP = _CHUNK // _LANES


def _sc_body(features_hbm, labels_hbm, centers_hbm, out_hbm,
             lbl_v, feats_v, rows_a, rows_b, acc_v, gsem, fsem, lsem):
    wid = lax.axis_index("s") * _NC + lax.axis_index("c")
    base = wid * _BPW

    pltpu.async_copy(labels_hbm.at[pl.ds(base, _BPW)], lbl_v, lsem).wait()
    fcopy = pltpu.async_copy(
        features_hbm.at[pl.ds(base * _FEAT_DIM, _BPW * _FEAT_DIM)],
        feats_v, fsem)
    bufs = (rows_a, rows_b)

    def issue_chunk(c, buf):
        def grp(g, _):
            lvec = lbl_v[pl.ds(c * _CHUNK + g * _LANES, _LANES)]
            off = lvec * _FEAT_DIM
            for j in range(_LANES):
                pltpu.async_copy(
                    centers_hbm.at[pl.ds(pl.multiple_of(off[j], _FEAT_DIM),
                                         _FEAT_DIM)],
                    buf.at[pl.ds((g * _LANES + j) * _FEAT_DIM, _FEAT_DIM)],
                    gsem.at[c % 2])
            return 0

        lax.fori_loop(0, _GRP, grp, 0)

    def drain_chunk(c, buf):
        def grp(g, _):
            for j in range(_LANES):
                pltpu.make_async_copy(
                    centers_hbm.at[pl.ds(0, _FEAT_DIM)],
                    buf.at[pl.ds((g * _LANES + j) * _FEAT_DIM, _FEAT_DIM)],
                    gsem.at[c % 2]).wait()
            return 0

        lax.fori_loop(0, _GRP, grp, 0)

    zero = jnp.zeros((_LANES,), jnp.float32)
    nacc = _FEAT_DIM // _LANES

    def compute_chunk(c, buf, accs):
        def body(i, accs):
            out = []
            for k in range(nacc):
                f = feats_v[pl.ds((c * _CHUNK + i) * _FEAT_DIM + k * _LANES,
                                  _LANES)]
                cc = buf[pl.ds(i * _FEAT_DIM + k * _LANES, _LANES)]
                d = f - cc
                out.append(accs[k] + d * d)
            return tuple(out)

        return lax.fori_loop(0, _CHUNK, body, accs)

    issue_chunk(0, bufs[0])
    accs = (zero,) * nacc
    fwaited = False
    for c in range(_NCHUNK):
        if c + 1 < _NCHUNK:
            issue_chunk(c + 1, bufs[(c + 1) % 2])
        drain_chunk(c, bufs[c % 2])
        if not fwaited:
            fcopy.wait()
            fwaited = True
        accs = compute_chunk(c, bufs[c % 2], accs)

    total = (accs[0] + accs[1]) + (accs[2] + accs[3])
    acc_v[...] = total * jnp.float32(1.0 / (_BATCH * _FEAT_DIM))
    pltpu.sync_copy(acc_v, out_hbm.at[wid])


@functools.partial(jax.jit, static_argnames=())
def kernel(features, labels, centers):
    mesh = plsc.VectorSubcoreMesh(core_axis_name="c", subcore_axis_name="s")
    partials = pl.kernel(
        _sc_body,
        mesh=mesh,
        out_type=jax.ShapeDtypeStruct((_NW, _LANES), jnp.float32),
        scratch_types=[
            pltpu.VMEM((_BPW,), jnp.int32),
            pltpu.VMEM((_BPW * _FEAT_DIM,), jnp.float32),
            pltpu.VMEM((_CHUNK * _FEAT_DIM,), jnp.float32),
            pltpu.VMEM((_CHUNK * _FEAT_DIM,), jnp.float32),
            pltpu.VMEM((_LANES,), jnp.float32),
            pltpu.SemaphoreType.DMA((2,)),
            pltpu.SemaphoreType.DMA,
            pltpu.SemaphoreType.DMA,
        ],
        compiler_params=pltpu.CompilerParams(use_tc_tiling_on_sc=False),
    )(features.reshape(_BATCH * _FEAT_DIM), labels.astype(jnp.int32),
      centers.reshape(_NUM_CLASSES * _FEAT_DIM))
    return jnp.sum(partials)


# speedup vs baseline: 1.2918x; 1.1311x over previous
import functools

import jax
import jax.numpy as jnp
from jax import lax
from jax.experimental import pallas as pl
from jax.experimental.pallas import tpu as pltpu
from jax.experimental.pallas import tpu_sc as plsc

_NUM_CLASSES = 100000
_FEAT_DIM = 64
_BATCH = 16384

_INFO = plsc.get_sparse_core_info()
_NC = _INFO.num_cores
_NS = _INFO.num_subcores
_LANES = _INFO.num_lanes
_NW = _NC * _NS
_BPW = _BATCH // _NW
_CHUNK = 128
_NCHUNK = _BPW // _CHUNK
_GRP = _CHUNK // _LANES

_BLK = 512
_NBLK = _BATCH // _BLK


def _sc_gather_body(labels_hbm, centers_hbm, out_hbm,
                    lbl_v, rows_a, rows_b, gsem, osem, lsem):
    wid = lax.axis_index("s") * _NC + lax.axis_index("c")
    base = wid * _BPW

    pltpu.async_copy(labels_hbm.at[pl.ds(base, _BPW)], lbl_v, lsem).wait()
    bufs = (rows_a, rows_b)

    def issue_chunk(c, buf):
        def grp(g, _):
            lvec = lbl_v[pl.ds(c * _CHUNK + g * _LANES, _LANES)]
            for j in range(_LANES):
                pltpu.async_copy(centers_hbm.at[lvec[j]],
                                 buf.at[g * _LANES + j], gsem.at[c % 2])
            return 0

        lax.fori_loop(0, _GRP, grp, 0)

    def drain_chunk(c, buf):
        def grp(g, _):
            for j in range(_LANES):
                pltpu.make_async_copy(
                    centers_hbm.at[0], buf.at[g * _LANES + j],
                    gsem.at[c % 2]).wait()
            return 0

        lax.fori_loop(0, _GRP, grp, 0)

    issue_chunk(0, bufs[0])
    ocopies = []
    for c in range(_NCHUNK):
        if c + 1 < _NCHUNK:
            issue_chunk(c + 1, bufs[(c + 1) % 2])
        drain_chunk(c, bufs[c % 2])
        ocopies.append(pltpu.async_copy(
            bufs[c % 2],
            out_hbm.at[pl.ds(base + c * _CHUNK, _CHUNK)], osem))
        if c >= 1:
            ocopies[c - 1].wait()
    ocopies[-1].wait()


def _tc_mse_body(f_ref, g_ref, o_ref):
    d = f_ref[...] - g_ref[...]
    o_ref[pl.ds(pl.program_id(0), 1), :] = jnp.sum(d * d, axis=0,
                                                   keepdims=True)


@functools.partial(jax.jit, static_argnames=())
def kernel(features, labels, centers):
    mesh = plsc.VectorSubcoreMesh(core_axis_name="c", subcore_axis_name="s")
    gathered = pl.kernel(
        _sc_gather_body,
        mesh=mesh,
        out_type=jax.ShapeDtypeStruct((_BATCH, _FEAT_DIM), jnp.float32),
        scratch_types=[
            pltpu.VMEM((_BPW,), jnp.int32),
            pltpu.VMEM((_CHUNK, _FEAT_DIM), jnp.float32),
            pltpu.VMEM((_CHUNK, _FEAT_DIM), jnp.float32),
            pltpu.SemaphoreType.DMA((2,)),
            pltpu.SemaphoreType.DMA,
            pltpu.SemaphoreType.DMA,
        ],
        compiler_params=pltpu.CompilerParams(use_tc_tiling_on_sc=True),
    )(labels.astype(jnp.int32), centers)

    partials = pl.pallas_call(
        _tc_mse_body,
        grid=(_NBLK,),
        out_shape=jax.ShapeDtypeStruct((_NBLK, _FEAT_DIM), jnp.float32),
        in_specs=[
            pl.BlockSpec((_BLK, _FEAT_DIM), lambda i: (i, 0)),
            pl.BlockSpec((_BLK, _FEAT_DIM), lambda i: (i, 0)),
        ],
        out_specs=pl.BlockSpec((_NBLK, _FEAT_DIM), lambda i: (0, 0)),
    )(features, gathered)
    return jnp.sum(partials) * jnp.float32(1.0 / (_BATCH * _FEAT_DIM))


# speedup vs baseline: 1.5260x; 1.1813x over previous
import functools

import jax
import jax.numpy as jnp
from jax import lax
from jax.experimental import pallas as pl
from jax.experimental.pallas import tpu as pltpu
from jax.experimental.pallas import tpu_sc as plsc

_NUM_CLASSES = 100000
_FEAT_DIM = 64
_BATCH = 16384

_INFO = plsc.get_sparse_core_info()
_NC = _INFO.num_cores
_NS = _INFO.num_subcores
_LANES = _INFO.num_lanes
_NW = _NC * _NS
_BPW = _BATCH // _NW
_CHUNK = 128
_NCHUNK = _BPW // _CHUNK
_GRP = _CHUNK // _LANES


def _sc_body(features_hbm, labels_hbm, centers_hbm, out_hbm,
             lbl_v, feats_v, rows_a, rows_b, acc_v, gsem, fsem, lsem):
    wid = lax.axis_index("s") * _NC + lax.axis_index("c")
    base = wid * _BPW

    pltpu.async_copy(labels_hbm.at[pl.ds(base, _BPW)], lbl_v, lsem).wait()
    fcopy = pltpu.async_copy(features_hbm.at[pl.ds(base, _BPW)],
                             feats_v, fsem)
    bufs = (rows_a, rows_b)

    def issue_chunk(c, buf):
        def grp(g, _):
            lvec = lbl_v[pl.ds(c * _CHUNK + g * _LANES, _LANES)]
            for j in range(_LANES):
                pltpu.async_copy(centers_hbm.at[lvec[j]],
                                 buf.at[g * _LANES + j], gsem.at[c % 2])
            return 0

        lax.fori_loop(0, _GRP, grp, 0)

    def drain_chunk(c, buf):
        def grp(g, _):
            for j in range(_LANES):
                pltpu.make_async_copy(
                    centers_hbm.at[0], buf.at[g * _LANES + j],
                    gsem.at[c % 2]).wait()
            return 0

        lax.fori_loop(0, _GRP, grp, 0)

    zero = jnp.zeros((_LANES,), jnp.float32)
    nacc = _FEAT_DIM // _LANES

    def compute_chunk(c, buf, accs):
        def body(i, accs):
            out = []
            for k in range(nacc):
                f = feats_v[c * _CHUNK + i, pl.ds(k * _LANES, _LANES)]
                cc = buf[i, pl.ds(k * _LANES, _LANES)]
                d = f - cc
                out.append(accs[k] + d * d)
            return tuple(out)

        return lax.fori_loop(0, _CHUNK, body, accs)

    issue_chunk(0, bufs[0])
    accs = (zero,) * nacc
    fwaited = False
    for c in range(_NCHUNK):
        if c + 1 < _NCHUNK:
            issue_chunk(c + 1, bufs[(c + 1) % 2])
        drain_chunk(c, bufs[c % 2])
        if not fwaited:
            fcopy.wait()
            fwaited = True
        accs = compute_chunk(c, bufs[c % 2], accs)

    total = (accs[0] + accs[1]) + (accs[2] + accs[3])
    acc_v[...] = total * jnp.float32(1.0 / (_BATCH * _FEAT_DIM))
    pltpu.sync_copy(acc_v, out_hbm.at[wid])


@functools.partial(jax.jit, static_argnames=())
def kernel(features, labels, centers):
    mesh = plsc.VectorSubcoreMesh(core_axis_name="c", subcore_axis_name="s")
    partials = pl.kernel(
        _sc_body,
        mesh=mesh,
        out_type=jax.ShapeDtypeStruct((_NW, _LANES), jnp.float32),
        scratch_types=[
            pltpu.VMEM((_BPW,), jnp.int32),
            pltpu.VMEM((_BPW, _FEAT_DIM), jnp.float32),
            pltpu.VMEM((_CHUNK, _FEAT_DIM), jnp.float32),
            pltpu.VMEM((_CHUNK, _FEAT_DIM), jnp.float32),
            pltpu.VMEM((_LANES,), jnp.float32),
            pltpu.SemaphoreType.DMA((2,)),
            pltpu.SemaphoreType.DMA,
            pltpu.SemaphoreType.DMA,
        ],
        compiler_params=pltpu.CompilerParams(use_tc_tiling_on_sc=True),
    )(features, labels.astype(jnp.int32), centers)
    return jnp.sum(partials)
